# baseline (speedup 1.0000x reference)
import functools

import jax
import jax.numpy as jnp
import numpy as np
from jax import lax
from jax.experimental import pallas as pl
from jax.experimental.pallas import tpu as pltpu
from jax.experimental.pallas import tpu_sc as plsc

N = 10000
D = 128
E = 320000

NC = 2
NS = 16
HD = D // NC
K = 128
CHUNKS = 158
EPT = CHUNKS * K
E_PAD = EPT * NS
HC = CHUNKS // 2
DEG_W = 16
N_PAD = 10112
RPT = N_PAD // NS

_PERM = np.zeros(HD, dtype=np.int32)
for _g in range(HD // 32):
    for _i in range(16):
        _PERM[32 * _g + 2 * _i] = 32 * _g + _i
        _PERM[32 * _g + 2 * _i + 1] = 32 * _g + 16 + _i


@functools.partial(
    pl.kernel,
    mesh=plsc.VectorSubcoreMesh(core_axis_name="c", subcore_axis_name="s"),
    compiler_params=pltpu.CompilerParams(use_tc_tiling_on_sc=False, needs_layout_passes=False),
    out_type=[
        jax.ShapeDtypeStruct((NC, N_PAD, HD), jnp.float32),
        jax.ShapeDtypeStruct((NS, N_PAD), jnp.float32),
    ],
    scratch_types=[
        pltpu.VMEM((CHUNKS, K), jnp.int32),
        pltpu.VMEM((CHUNKS, K), jnp.int32),
        pltpu.VMEM((2, K, HD), jnp.bfloat16),
        pltpu.VMEM((K, HD), jnp.float32),
        pltpu.VMEM((N_PAD,), jnp.float32),
        pltpu.VMEM_SHARED((N_PAD, HD), jnp.bfloat16),
        pltpu.VMEM_SHARED((N_PAD, HD), jnp.float32),
        pltpu.SemaphoreType.DMA,
        pltpu.SemaphoreType.DMA,
        pltpu.SemaphoreType.DMA,
    ],
)
def _sc_propagate(y_hbm, src_hbm, dst_hbm, zacc_hbm,
                  acc_out, deg_out,
                  src_v, dst_v, rows_bf, rows_f32, degl,
                  y_sh, acc_sh,
                  sg0, sg1, ss0):
    cid = lax.axis_index("c")
    sid = lax.axis_index("s")
    sg = (sg0, sg1)

    pltpu.sync_copy(zacc_hbm, acc_sh.at[pl.ds(sid * RPT, RPT)])
    pltpu.sync_copy(y_hbm.at[cid, pl.ds(sid * RPT, RPT)],
                    y_sh.at[pl.ds(sid * RPT, RPT)])

    def _zero_deg(i, carry):
        degl[pl.ds(i * 16, 16)] = jnp.zeros((16,), jnp.float32)
        return carry
    lax.fori_loop(0, N_PAD // 16, _zero_deg, 0)

    pltpu.sync_copy(src_hbm.at[sid], src_v)
    pltpu.sync_copy(dst_hbm.at[sid], dst_v)

    plsc.subcore_barrier()

    def _gather(j, b):
        return pltpu.make_async_copy(
            y_sh.at[src_v.at[j]], rows_bf.at[b], sg[b])

    def _scatter(j):
        return pltpu.make_async_copy(
            rows_f32, acc_sh.at[dst_v.at[j]], ss0)

    def _convert(b):
        def _rows(ro, carry):
            base = ro * 8
            for rr in range(8):
                r = base + rr
                for g in range(HD // 32):
                    ab = rows_bf[b, r, pl.ds(32 * g, 32)]
                    lo, hi = plsc.unpack(
                        ab, format=plsc.PackFormat.INTERLEAVED)
                    rows_f32[r, pl.ds(32 * g, 16)] = lo
                    rows_f32[r, pl.ds(32 * g + 16, 16)] = hi
            return carry
        lax.fori_loop(0, K // 8, _rows, 0)

    _gather(0, 0).start()

    def _pair(jo, carry):
        for b in (0, 1):
            j = 2 * jo + b
            _gather(j, b).wait()

            def _next_gather():
                _gather(j + 1, 1 - b).start()

            if b == 0:
                _next_gather()
            else:
                @pl.when(jo < HC - 1)
                def _():
                    _next_gather()

            @pl.when(cid == 0)
            def _():
                for q in range(K // 16):
                    idx = dst_v[j, pl.ds(16 * q, 16)]
                    plsc.addupdate_scatter(
                        degl, [idx], jnp.ones((16,), jnp.float32))

            if b == 0:
                @pl.when(jo >= 1)
                def _():
                    _scatter(j - 1).wait()
            else:
                _scatter(j - 1).wait()

            _convert(b)
            _scatter(j).start(add=True)
        return carry
    lax.fori_loop(0, HC, _pair, 0)

    _scatter(CHUNKS - 1).wait()

    plsc.subcore_barrier()

    pltpu.sync_copy(acc_sh.at[pl.ds(sid * RPT, RPT)],
                    acc_out.at[cid, pl.ds(sid * RPT, RPT)])

    @pl.when(cid == 0)
    def _():
        pltpu.sync_copy(degl, deg_out.at[sid])


_BLK = 400


def _combine_body(acc_ref, deg_ref, out_ref):
    d = jnp.maximum(jnp.sum(deg_ref[...], axis=1), 1.0).reshape(_BLK, 1)
    out_ref[:, 0:HD] = acc_ref[0] / d
    out_ref[:, HD:D] = acc_ref[1] / d


def _combine(acc_p, deg_p):
    return pl.pallas_call(
        _combine_body,
        grid=(N // _BLK,),
        in_specs=[
            pl.BlockSpec((NC, _BLK, HD), lambda i: (0, i, 0)),
            pl.BlockSpec((_BLK, NS), lambda i: (i, 0)),
        ],
        out_specs=pl.BlockSpec((_BLK, D), lambda i: (i, 0)),
        out_shape=jax.ShapeDtypeStruct((N, D), jnp.float32),
    )(acc_p, deg_p)


@jax.jit
def kernel(y_author, y_paper, y_venue, edge_author_paper, edge_paper_author,
           edge_paper_venue, edge_venue_paper):
    src = edge_paper_author[0].astype(jnp.int32)
    dst = edge_paper_author[1].astype(jnp.int32)
    pad = E_PAD - E
    src = jnp.concatenate([src, jnp.zeros((pad,), jnp.int32)])
    dst = jnp.concatenate([dst, jnp.full((pad,), N, jnp.int32)])
    src = src.reshape(NS, CHUNKS, K)
    dst = dst.reshape(NS, CHUNKS, K)
    y_halves = jnp.stack([y_paper[:, :HD], y_paper[:, HD:]])
    y_bf = jnp.zeros((NC, N_PAD, HD), jnp.bfloat16)
    y_bf = y_bf.at[:, :N].set(y_halves[:, :, _PERM].astype(jnp.bfloat16))
    zacc = jnp.zeros((RPT, HD), jnp.float32)
    acc_p, deg_p = _sc_propagate(y_bf, src, dst, zacc)
    return _combine(acc_p, deg_p.T)


# speedup vs baseline: 1.5132x; 1.5132x over previous
import functools

import jax
import jax.numpy as jnp
from jax import lax
from jax.experimental import pallas as pl
from jax.experimental.pallas import tpu as pltpu
from jax.experimental.pallas import tpu_sc as plsc

N = 10000
D = 128
E = 320000

NC = 2
NS = 16
HD = D // NC
K = 128
NB = 4
IB = 8
CHUNKS = 160
NBLK = CHUNKS // IB
EPT = CHUNKS * K
E_PAD = EPT * NS
DEG_W = 16
N_PAD = 10112
RPT = N_PAD // NS


@functools.partial(
    pl.kernel,
    mesh=plsc.VectorSubcoreMesh(core_axis_name="c", subcore_axis_name="s"),
    compiler_params=pltpu.CompilerParams(use_tc_tiling_on_sc=False),
    out_type=[
        jax.ShapeDtypeStruct((NC, N_PAD, HD), jnp.float32),
        jax.ShapeDtypeStruct((N_PAD, DEG_W), jnp.float32),
    ],
    scratch_types=[
        pltpu.VMEM((NB, K, HD), jnp.float32),
        pltpu.VMEM((K, DEG_W), jnp.float32),
        pltpu.VMEM((2, IB, K), jnp.int32),
        pltpu.VMEM((2, IB, K), jnp.int32),
        pltpu.VMEM_SHARED((N_PAD, HD), jnp.float32),
        pltpu.VMEM_SHARED((N_PAD, HD), jnp.float32),
        pltpu.VMEM_SHARED((N_PAD, DEG_W), jnp.float32),
        pltpu.SemaphoreType.DMA,
        pltpu.SemaphoreType.DMA,
        pltpu.SemaphoreType.DMA,
        pltpu.SemaphoreType.DMA,
        pltpu.SemaphoreType.DMA,
        pltpu.SemaphoreType.DMA,
        pltpu.SemaphoreType.DMA,
        pltpu.SemaphoreType.DMA,
        pltpu.SemaphoreType.DMA,
        pltpu.SemaphoreType.DMA,
        pltpu.SemaphoreType.DMA,
        pltpu.SemaphoreType.DMA,
    ],
)
def _sc_propagate(y_hbm, src_hbm, dst_hbm, zacc_hbm,
                  acc_out, deg_out,
                  rows, ones, src_blk, dst_blk, y_sh, acc_sh, deg_sh,
                  sg0, sg1, sg2, sg3, ss0, ss1, ss2, ss3, sd0, sd1,
                  si0, si1):
    cid = lax.axis_index("c")
    sid = lax.axis_index("s")
    sg = (sg0, sg1, sg2, sg3)
    ss = (ss0, ss1, ss2, ss3)
    sd = (sd0, sd1)

    pltpu.sync_copy(y_hbm.at[cid, pl.ds(sid * RPT, RPT)],
                    y_sh.at[pl.ds(sid * RPT, RPT)])

    pltpu.sync_copy(zacc_hbm, acc_sh.at[pl.ds(sid * RPT, RPT)])

    @pl.when(cid == 0)
    def _():
        pltpu.sync_copy(zacc_hbm.at[:, pl.ds(0, DEG_W)],
                        deg_sh.at[pl.ds(sid * RPT, RPT)])

    ov = jnp.ones((16,), jnp.float32)

    def _fill_ones(i, carry):
        ones[i, pl.ds(0, 16)] = ov
        return carry
    lax.fori_loop(0, K, _fill_ones, 0)

    plsc.subcore_barrier()

    def _gather(pb, row, rb):
        return pltpu.make_async_copy(
            y_sh.at[src_blk.at[pb, row]], rows.at[rb], sg[rb])

    def _scatter(pb, row, rb):
        return pltpu.make_async_copy(
            rows.at[rb], acc_sh.at[dst_blk.at[pb, row]], ss[rb])

    def _deg_scatter(pb, row, p):
        return pltpu.make_async_copy(
            ones, deg_sh.at[dst_blk.at[pb, row]], sd[p])

    def _idx_copies(nbn, pb):
        return (
            pltpu.make_async_copy(
                src_hbm.at[sid, pl.ds(nbn * IB, IB)], src_blk.at[pb], si0),
            pltpu.make_async_copy(
                dst_hbm.at[sid, pl.ds(nbn * IB, IB)], dst_blk.at[pb], si1),
        )

    for c in _idx_copies(0, 0):
        c.start()
    for c in _idx_copies(0, 0):
        c.wait()
    _gather(0, 0, 0).start()
    _gather(0, 1, 1).start()

    def _block(nb, carry):
        cur = lax.rem(nb, 2)
        oth = 1 - cur
        for b in range(IB):
            rb = b % NB
            _gather(cur, b, rb).wait()

            def _wait_sc():
                _scatter(cur, b, (b + 2) % NB).wait()

            if b >= 2:
                _wait_sc()
            else:
                @pl.when(nb >= 1)
                def _():
                    _wait_sc()

            @pl.when(cid == 0)
            def _():
                def _wait_dg():
                    _deg_scatter(cur, b, b % 2).wait()
                if b >= 2:
                    _wait_dg()
                else:
                    @pl.when(nb >= 1)
                    def _():
                        _wait_dg()
                _deg_scatter(cur, b, b % 2).start(add=True)

            if b == 2:
                @pl.when(nb < NBLK - 1)
                def _():
                    for c in _idx_copies(nb + 1, oth):
                        c.start()

            if b <= IB - 3:
                _gather(cur, b + 2, (b + 2) % NB).start()
            else:
                if b == IB - 2:
                    @pl.when(nb < NBLK - 1)
                    def _():
                        for c in _idx_copies(nb + 1, oth):
                            c.wait()

                @pl.when(nb < NBLK - 1)
                def _():
                    _gather(oth, b - (IB - 2), (b + 2) % NB).start()

            _scatter(cur, b, rb).start(add=True)
        return carry
    lax.fori_loop(0, NBLK, _block, 0)

    _scatter(0, 0, (CHUNKS - 2) % NB).wait()
    _scatter(0, 0, (CHUNKS - 1) % NB).wait()

    @pl.when(cid == 0)
    def _():
        _deg_scatter(0, 0, 0).wait()
        _deg_scatter(0, 0, 1).wait()

    plsc.subcore_barrier()

    pltpu.sync_copy(acc_sh.at[pl.ds(sid * RPT, RPT)],
                    acc_out.at[cid, pl.ds(sid * RPT, RPT)])

    @pl.when(cid == 0)
    def _():
        pltpu.sync_copy(deg_sh.at[pl.ds(sid * RPT, RPT)],
                        deg_out.at[pl.ds(sid * RPT, RPT)])


_BLK = 400


def _combine_body(acc_ref, deg_ref, out_ref):
    d = jnp.maximum(jnp.sum(deg_ref[...], axis=1) * (1.0 / DEG_W),
                    1.0).reshape(_BLK, 1)
    out_ref[:, 0:HD] = acc_ref[0] / d
    out_ref[:, HD:D] = acc_ref[1] / d


def _combine(acc_p, deg_p):
    return pl.pallas_call(
        _combine_body,
        grid=(N // _BLK,),
        in_specs=[
            pl.BlockSpec((NC, _BLK, HD), lambda i: (0, i, 0)),
            pl.BlockSpec((_BLK, DEG_W), lambda i: (i, 0)),
        ],
        out_specs=pl.BlockSpec((_BLK, D), lambda i: (i, 0)),
        out_shape=jax.ShapeDtypeStruct((N, D), jnp.float32),
    )(acc_p, deg_p)


@jax.jit
def kernel(y_author, y_paper, y_venue, edge_author_paper, edge_paper_author,
           edge_paper_venue, edge_venue_paper):
    src = edge_paper_author[0].astype(jnp.int32)
    dst = edge_paper_author[1].astype(jnp.int32)
    pad = E_PAD - E
    src = jnp.concatenate([src, jnp.zeros((pad,), jnp.int32)])
    dst = jnp.concatenate([dst, jnp.full((pad,), N, jnp.int32)])
    src = src.reshape(NS, CHUNKS, K)
    dst = dst.reshape(NS, CHUNKS, K)
    y_halves = jnp.stack([y_paper[:, :HD], y_paper[:, HD:]])
    y_st = jnp.zeros((NC, N_PAD, HD), jnp.float32)
    y_st = y_st.at[:, :N].set(y_halves)
    zacc = jnp.zeros((RPT, HD), jnp.float32)
    acc_p, deg_p = _sc_propagate(y_st, src, dst, zacc)
    return _combine(acc_p, deg_p)
